# baseline (speedup 1.0000x reference)
import functools

import jax
import jax.numpy as jnp
from jax import lax
from jax.experimental import pallas as pl
from jax.experimental.pallas import tpu as pltpu
from jax.experimental.pallas import tpu_sc as plsc

HIDDEN = 64
PADH = 128
BLK = 128
NW = 32
L = 16


def _gather_kernel(seq: int, batch: int, nrows: int):
    mesh = plsc.VectorSubcoreMesh(core_axis_name="c", subcore_axis_name="s")

    @functools.partial(
        pl.kernel,
        mesh=mesh,
        out_type=jax.ShapeDtypeStruct((seq, HIDDEN, batch), jnp.float32),
        scratch_types=[
            pltpu.VMEM((seq, BLK), jnp.int32),
            pltpu.VMEM((2 * BLK, PADH), jnp.float32),
            pltpu.VMEM((2 * BLK, PADH), jnp.float32),
            pltpu.VMEM((HIDDEN, BLK), jnp.float32),
            pltpu.VMEM((HIDDEN, BLK), jnp.float32),
            pltpu.SemaphoreType.DMA,
            pltpu.SemaphoreType.DMA,
            pltpu.SemaphoreType.DMA,
        ],
        compiler_params=pltpu.CompilerParams(
            use_tc_tiling_on_sc=True, needs_layout_passes=False,
            disable_bounds_checks=True),
    )
    def k(ids_hbm, table_hbm, out_hbm, idx_v, g0, g1, t0, t1, sg0, sg1, st):
        wid = lax.axis_index("s") * 2 + lax.axis_index("c")
        i0 = wid * BLK
        gbanks = (g0, g1)
        tbanks = (t0, t1)
        gsems = (sg0, sg1)

        pltpu.sync_copy(ids_hbm.at[:, pl.ds(i0, BLK)], idx_v)

        def fire_pair(jp, p):
            for h in range(2):
                pltpu.async_copy(
                    table_hbm.at[idx_v.at[2 * jp + h]],
                    gbanks[p].at[pl.ds(h * BLK, BLK)], gsems[p])

        def drain_pair(p):
            for h in range(2):
                pltpu.make_async_copy(
                    table_hbm.at[idx_v.at[0]],
                    gbanks[p].at[pl.ds(h * BLK, BLK)], gsems[p]).wait()

        def transpose_block(p, h, t):
            g = gbanks[p]
            rows = [lax.iota(jnp.int32, L) + (h * BLK + c * L)
                    for c in range(BLK // L)]

            def _(hh, carry):
                cols = jnp.full((L,), hh, jnp.int32)
                for c in range(BLK // L):
                    t[hh, pl.ds(c * L, L)] = plsc.load_gather(
                        g, [rows[c], cols])
                return carry
            lax.fori_loop(0, HIDDEN, _, 0)

        def fire_out(j, t):
            pltpu.async_copy(t, out_hbm.at[j, :, pl.ds(i0, BLK)], st)

        def drain_out(t):
            pltpu.make_async_copy(
                t, out_hbm.at[0, :, pl.ds(i0, BLK)], st).wait()

        fire_pair(0, 0)

        def body(jj, carry):
            for p in range(2):
                jp = 2 * jj + p
                drain_pair(p)

                @pl.when(jp + 1 < seq // 2)
                def _():
                    fire_pair(jp + 1, 1 - p)

                for h in range(2):
                    @pl.when(jp >= 1)
                    def _():
                        drain_out(tbanks[h])

                    transpose_block(p, h, tbanks[h])
                    fire_out(2 * jp + h, tbanks[h])
            return carry

        lax.fori_loop(0, seq // 4, body, 0)
        drain_out(tbanks[0])
        drain_out(tbanks[1])

    return k


def kernel(ids, table):
    b, s, _ = ids.shape
    idx_t = jnp.transpose(ids[:, :, 0]).astype(jnp.int32)
    table_p = jnp.pad(table, ((0, 0), (0, PADH - HIDDEN)))
    out_p = _gather_kernel(s, b, table.shape[0])(idx_t, table_p)
    return jnp.transpose(out_p, (2, 0, 1))


# speedup vs baseline: 1.7417x; 1.7417x over previous
import functools

import jax
import jax.numpy as jnp
from jax import lax
from jax.experimental import pallas as pl
from jax.experimental.pallas import tpu as pltpu
from jax.experimental.pallas import tpu_sc as plsc

HIDDEN = 64
PADH = 128
CHUNK = 128
NBANK = 4
NW = 32


def _gather_kernel(nrows_out: int, nrows_tab: int):
    per_w = nrows_out // NW
    nchunk = per_w // CHUNK
    mesh = plsc.VectorSubcoreMesh(core_axis_name="c", subcore_axis_name="s")

    @functools.partial(
        pl.kernel,
        mesh=mesh,
        out_type=jax.ShapeDtypeStruct((nrows_out, PADH), jnp.float32),
        scratch_types=[
            pltpu.VMEM((nchunk, CHUNK), jnp.int32),
            pltpu.VMEM((NBANK, CHUNK, PADH), jnp.float32),
            pltpu.SemaphoreType.DMA,
            pltpu.SemaphoreType.DMA,
        ],
        compiler_params=pltpu.CompilerParams(
            use_tc_tiling_on_sc=True, needs_layout_passes=False,
            disable_bounds_checks=True),
    )
    def k(ids_hbm, table_hbm, out_hbm, idx_v, banks, gsem, wsem):
        wid = lax.axis_index("s") * 2 + lax.axis_index("c")
        row0 = wid * per_w

        pltpu.sync_copy(ids_hbm.at[pl.ds(wid * nchunk, nchunk)], idx_v)

        def fire_gather(j, b):
            pltpu.async_copy(table_hbm.at[idx_v.at[j]], banks.at[b], gsem)

        def drain_gather(b):
            pltpu.make_async_copy(
                table_hbm.at[idx_v.at[0]], banks.at[b], gsem).wait()

        def fire_write(j, b):
            pltpu.async_copy(
                banks.at[b], out_hbm.at[pl.ds(row0 + j * CHUNK, CHUNK)], wsem)

        def drain_write(b):
            pltpu.make_async_copy(
                banks.at[b], out_hbm.at[pl.ds(row0, CHUNK)], wsem).wait()

        for b in range(NBANK - 1):
            fire_gather(b, b)

        def body(g, carry):
            for b in range(NBANK):
                j = g * NBANK + b
                drain_gather(b)
                fire_write(j, b)
                nb = (b + NBANK - 1) % NBANK

                @pl.when(j + NBANK - 1 < nchunk)
                def _():
                    @pl.when(j >= 1)
                    def _():
                        drain_write(nb)
                    fire_gather(j + NBANK - 1, nb)
            return carry

        lax.fori_loop(0, nchunk // NBANK, body, 0)
        for _ in range(NBANK):
            drain_write(0)

    return k


def kernel(ids, table):
    b, s, _ = ids.shape
    n = b * s
    idx = jnp.reshape(ids[:, :, 0].astype(jnp.int32), (n // CHUNK, CHUNK))
    table_p = jnp.pad(table, ((0, 0), (0, PADH - HIDDEN)))
    out = _gather_kernel(n, table.shape[0])(idx, table_p)
    return jnp.reshape(out[:, :HIDDEN], (b, s, HIDDEN))
